# baseline (speedup 1.0000x reference)
import functools

import jax
import jax.numpy as jnp
from jax.experimental import pallas as pl
from jax.experimental.pallas import tpu as pltpu

_BLK = 800


def _nn_kernel(n_blocks, d1t_ref, d2_ref, d1sq_ref, d2sq_ref, dist_ref,
               idx_ref, minval_ref, minidx_ref):
    i = pl.program_id(0)
    d1t = d1t_ref[...]
    d2b = d2_ref[...]
    n1 = d1t.shape[1]

    mm = jax.lax.dot_general(
        d2b, d1t, (((1,), (0,)), ((), ())),
        preferred_element_type=jnp.float32)
    s = (d2sq_ref[...] + d1sq_ref[...]) - 2.0 * mm

    bmin = jnp.min(s, axis=0, keepdims=True)
    row = jax.lax.broadcasted_iota(jnp.int32, s.shape, 0) + i * _BLK
    big = jnp.int32(2**31 - 1)
    bidx = jnp.min(jnp.where(s == bmin, row, big), axis=0, keepdims=True)

    @pl.when(i == 0)
    def _():
        minval_ref[...] = bmin
        minidx_ref[...] = bidx

    @pl.when(i > 0)
    def _():
        better = bmin < minval_ref[...]
        minval_ref[...] = jnp.where(better, bmin, minval_ref[...])
        minidx_ref[...] = jnp.where(better, bidx, minidx_ref[...])

    @pl.when(i == n_blocks - 1)
    def _():
        dist_ref[...] = jnp.sqrt(jnp.clip(minval_ref[...], 0.0, None))
        idx_ref[...] = minidx_ref[...]


def kernel(desc1, desc2):
    n1, dim = desc1.shape
    n2 = desc2.shape[0]
    assert n2 % _BLK == 0
    n_blocks = n2 // _BLK

    d1t = desc1.T
    d1sq = jnp.sum(desc1 ** 2, axis=1)[None, :]
    d2sq = jnp.sum(desc2 ** 2, axis=1, keepdims=True)

    dists_t, idxs_t = pl.pallas_call(
        functools.partial(_nn_kernel, n_blocks),
        grid=(n_blocks,),
        in_specs=[
            pl.BlockSpec((dim, n1), lambda i: (0, 0)),
            pl.BlockSpec((_BLK, dim), lambda i: (i, 0)),
            pl.BlockSpec((1, n1), lambda i: (0, 0)),
            pl.BlockSpec((_BLK, 1), lambda i: (i, 0)),
        ],
        out_specs=[
            pl.BlockSpec((1, n1), lambda i: (0, 0)),
            pl.BlockSpec((1, n1), lambda i: (0, 0)),
        ],
        out_shape=[
            jax.ShapeDtypeStruct((1, n1), jnp.float32),
            jax.ShapeDtypeStruct((1, n1), jnp.int32),
        ],
        scratch_shapes=[
            pltpu.VMEM((1, n1), jnp.float32),
            pltpu.VMEM((1, n1), jnp.int32),
        ],
        compiler_params=pltpu.CompilerParams(
            dimension_semantics=("arbitrary",)),
    )(d1t, desc2, d1sq, d2sq)

    match_dists = dists_t.reshape(n1, 1)
    rows = jnp.arange(n1, dtype=jnp.int32)[:, None]
    matches_idxs = jnp.concatenate([rows, idxs_t.reshape(n1, 1)], axis=1)
    return match_dists, matches_idxs


# speedup vs baseline: 1.1657x; 1.1657x over previous
import functools

import jax
import jax.numpy as jnp
from jax.experimental import pallas as pl
from jax.experimental.pallas import tpu as pltpu

_BLK = 800


def _nn_kernel(n_blocks, d1t_ref, d2_ref, d1sq_ref, d2sq_ref, dist_ref,
               idx_ref, minval_ref, minidx_ref):
    i = pl.program_id(0)
    d1t = d1t_ref[...]
    d2b = d2_ref[...]
    n1 = d1t.shape[1]
    blk = d2b.shape[0]

    mm = jax.lax.dot_general(
        d2b, d1t, (((1,), (0,)), ((), ())),
        preferred_element_type=jnp.float32)
    d2sq = jnp.transpose(d2sq_ref[0], (1, 0))
    s = (d2sq + d1sq_ref[...]) + mm

    bmin = jnp.min(s, axis=0, keepdims=True)
    row = jax.lax.broadcasted_iota(jnp.int32, s.shape, 0) + i * _BLK
    big = jnp.int32(2**31 - 1)
    bidx = jnp.min(jnp.where(s == bmin, row, big), axis=0, keepdims=True)

    @pl.when(i == 0)
    def _():
        minval_ref[...] = bmin
        minidx_ref[...] = bidx

    @pl.when(i > 0)
    def _():
        better = bmin < minval_ref[...]
        minval_ref[...] = jnp.where(better, bmin, minval_ref[...])
        minidx_ref[...] = jnp.where(better, bidx, minidx_ref[...])

    @pl.when(i == n_blocks - 1)
    def _():
        dist_ref[...] = jnp.sqrt(jnp.clip(minval_ref[...], 0.0, None))
        idx_ref[...] = minidx_ref[...]


def kernel(desc1, desc2):
    n1, dim = desc1.shape
    n2 = desc2.shape[0]
    assert n2 % _BLK == 0
    n_blocks = n2 // _BLK

    d1t = desc1.T * jnp.float32(-2.0)
    d1sq = jnp.sum(desc1 ** 2, axis=1)[None, :]
    d2sq = jnp.sum(desc2 ** 2, axis=1).reshape(n_blocks, 1, _BLK)

    dists_t, idxs_t = pl.pallas_call(
        functools.partial(_nn_kernel, n_blocks),
        grid=(n_blocks,),
        in_specs=[
            pl.BlockSpec((dim, n1), lambda i: (0, 0)),
            pl.BlockSpec((_BLK, dim), lambda i: (i, 0)),
            pl.BlockSpec((1, n1), lambda i: (0, 0)),
            pl.BlockSpec((1, 1, _BLK), lambda i: (i, 0, 0)),
        ],
        out_specs=[
            pl.BlockSpec((1, n1), lambda i: (0, 0)),
            pl.BlockSpec((1, n1), lambda i: (0, 0)),
        ],
        out_shape=[
            jax.ShapeDtypeStruct((1, n1), jnp.float32),
            jax.ShapeDtypeStruct((1, n1), jnp.int32),
        ],
        scratch_shapes=[
            pltpu.VMEM((1, n1), jnp.float32),
            pltpu.VMEM((1, n1), jnp.int32),
        ],
        compiler_params=pltpu.CompilerParams(
            dimension_semantics=("arbitrary",)),
    )(d1t, desc2, d1sq, d2sq)

    match_dists = dists_t.reshape(n1, 1)
    rows = jnp.arange(n1, dtype=jnp.int32)[:, None]
    matches_idxs = jnp.concatenate([rows, idxs_t.reshape(n1, 1)], axis=1)
    return match_dists, matches_idxs


# speedup vs baseline: 1.4751x; 1.2654x over previous
import functools

import jax
import jax.numpy as jnp
from jax.experimental import pallas as pl
from jax.experimental.pallas import tpu as pltpu

_BLK = 800


def _nn_kernel(n_blocks, d1t_ref, d2_ref, d1sq_ref, d2sq_ref, dist_ref,
               idx_ref, minval_ref, minidx_ref):
    i = pl.program_id(0)
    d1t = d1t_ref[...]
    d2b = d2_ref[...]
    n1 = d1t.shape[1]
    blk = d2b.shape[0]

    mm = jax.lax.dot_general(
        d2b, d1t, (((1,), (0,)), ((), ())),
        preferred_element_type=jnp.float32)
    d2sq = jnp.transpose(d2sq_ref[0], (1, 0))
    s = (d2sq + d1sq_ref[...]) + mm

    s4 = s.reshape(blk // 8, 8, n1)
    bmin8 = jnp.min(s4, axis=0)
    r8 = jnp.argmin(s4, axis=0).astype(jnp.int32)
    sub = jax.lax.broadcasted_iota(jnp.int32, (8, n1), 0)
    idx8 = r8 * 8 + sub + i * _BLK
    big = jnp.int32(2**31 - 1)
    bmin = jnp.min(bmin8, axis=0, keepdims=True)
    bidx = jnp.min(jnp.where(bmin8 == bmin, idx8, big), axis=0,
                   keepdims=True)

    @pl.when(i == 0)
    def _():
        minval_ref[...] = bmin
        minidx_ref[...] = bidx

    @pl.when(i > 0)
    def _():
        better = bmin < minval_ref[...]
        minval_ref[...] = jnp.where(better, bmin, minval_ref[...])
        minidx_ref[...] = jnp.where(better, bidx, minidx_ref[...])

    @pl.when(i == n_blocks - 1)
    def _():
        dist_ref[...] = jnp.sqrt(jnp.clip(minval_ref[...], 0.0, None))
        idx_ref[...] = minidx_ref[...]


def kernel(desc1, desc2):
    n1, dim = desc1.shape
    n2 = desc2.shape[0]
    assert n2 % _BLK == 0
    n_blocks = n2 // _BLK

    d1t = desc1.T * jnp.float32(-2.0)
    d1sq = jnp.sum(desc1 ** 2, axis=1)[None, :]
    d2sq = jnp.sum(desc2 ** 2, axis=1).reshape(n_blocks, 1, _BLK)

    dists_t, idxs_t = pl.pallas_call(
        functools.partial(_nn_kernel, n_blocks),
        grid=(n_blocks,),
        in_specs=[
            pl.BlockSpec((dim, n1), lambda i: (0, 0)),
            pl.BlockSpec((_BLK, dim), lambda i: (i, 0)),
            pl.BlockSpec((1, n1), lambda i: (0, 0)),
            pl.BlockSpec((1, 1, _BLK), lambda i: (i, 0, 0)),
        ],
        out_specs=[
            pl.BlockSpec((1, n1), lambda i: (0, 0)),
            pl.BlockSpec((1, n1), lambda i: (0, 0)),
        ],
        out_shape=[
            jax.ShapeDtypeStruct((1, n1), jnp.float32),
            jax.ShapeDtypeStruct((1, n1), jnp.int32),
        ],
        scratch_shapes=[
            pltpu.VMEM((1, n1), jnp.float32),
            pltpu.VMEM((1, n1), jnp.int32),
        ],
        compiler_params=pltpu.CompilerParams(
            dimension_semantics=("arbitrary",)),
    )(d1t, desc2, d1sq, d2sq)

    match_dists = dists_t.reshape(n1, 1)
    rows = jnp.arange(n1, dtype=jnp.int32)[:, None]
    matches_idxs = jnp.concatenate([rows, idxs_t.reshape(n1, 1)], axis=1)
    return match_dists, matches_idxs


# speedup vs baseline: 1.8208x; 1.2344x over previous
import functools

import jax
import jax.numpy as jnp
from jax.experimental import pallas as pl
from jax.experimental.pallas import tpu as pltpu

_BLK = 10000


def _nn_kernel(n_blocks, d1t_ref, d2_ref, d1sq_ref, d2sq_ref, dist_ref,
               idx_ref, minval_ref, minidx_ref):
    i = pl.program_id(0)
    d1t = d1t_ref[...]
    d2b = d2_ref[...]
    n1 = d1t.shape[1]
    blk = d2b.shape[0]

    mm = jax.lax.dot_general(
        d2b, d1t, (((1,), (0,)), ((), ())),
        preferred_element_type=jnp.float32)
    d2sq = jnp.transpose(d2sq_ref[0], (1, 0))
    s = (d2sq + d1sq_ref[...]) + mm

    s4 = s.reshape(blk // 8, 8, n1)
    bmin8 = jnp.min(s4, axis=0)
    r8 = jnp.argmin(s4, axis=0).astype(jnp.int32)
    sub = jax.lax.broadcasted_iota(jnp.int32, (8, n1), 0)
    idx8 = r8 * 8 + sub + i * _BLK
    big = jnp.int32(2**31 - 1)
    bmin = jnp.min(bmin8, axis=0, keepdims=True)
    bidx = jnp.min(jnp.where(bmin8 == bmin, idx8, big), axis=0,
                   keepdims=True)

    @pl.when(i == 0)
    def _():
        minval_ref[...] = bmin
        minidx_ref[...] = bidx

    @pl.when(i > 0)
    def _():
        better = bmin < minval_ref[...]
        minval_ref[...] = jnp.where(better, bmin, minval_ref[...])
        minidx_ref[...] = jnp.where(better, bidx, minidx_ref[...])

    @pl.when(i == n_blocks - 1)
    def _():
        dist_ref[...] = jnp.sqrt(jnp.clip(minval_ref[...], 0.0, None))
        idx_ref[...] = minidx_ref[...]


def kernel(desc1, desc2):
    n1, dim = desc1.shape
    n2 = desc2.shape[0]
    assert n2 % _BLK == 0
    n_blocks = n2 // _BLK

    d1t = desc1.T * jnp.float32(-2.0)
    d1sq = jnp.sum(desc1 ** 2, axis=1)[None, :]
    d2sq = jnp.sum(desc2 ** 2, axis=1).reshape(n_blocks, 1, _BLK)

    dists_t, idxs_t = pl.pallas_call(
        functools.partial(_nn_kernel, n_blocks),
        grid=(n_blocks,),
        in_specs=[
            pl.BlockSpec((dim, n1), lambda i: (0, 0)),
            pl.BlockSpec((_BLK, dim), lambda i: (i, 0)),
            pl.BlockSpec((1, n1), lambda i: (0, 0)),
            pl.BlockSpec((1, 1, _BLK), lambda i: (i, 0, 0)),
        ],
        out_specs=[
            pl.BlockSpec((1, n1), lambda i: (0, 0)),
            pl.BlockSpec((1, n1), lambda i: (0, 0)),
        ],
        out_shape=[
            jax.ShapeDtypeStruct((1, n1), jnp.float32),
            jax.ShapeDtypeStruct((1, n1), jnp.int32),
        ],
        scratch_shapes=[
            pltpu.VMEM((1, n1), jnp.float32),
            pltpu.VMEM((1, n1), jnp.int32),
        ],
        compiler_params=pltpu.CompilerParams(
            dimension_semantics=("arbitrary",)),
    )(d1t, desc2, d1sq, d2sq)

    match_dists = dists_t.reshape(n1, 1)
    rows = jnp.arange(n1, dtype=jnp.int32)[:, None]
    matches_idxs = jnp.concatenate([rows, idxs_t.reshape(n1, 1)], axis=1)
    return match_dists, matches_idxs


# speedup vs baseline: 2.0419x; 1.1215x over previous
import functools

import jax
import jax.numpy as jnp
from jax.experimental import pallas as pl
from jax.experimental.pallas import tpu as pltpu

_BLK = 10000


def _nn_kernel(n_blocks, d1t_ref, d2_ref, d1sq_ref, dist_ref,
               idx_ref, minval_ref, minidx_ref):
    i = pl.program_id(0)
    d1t = d1t_ref[...]
    d2b = d2_ref[...]
    n1 = d1t.shape[1]
    blk = d2b.shape[0]

    mm = jax.lax.dot_general(
        d2b, d1t, (((1,), (0,)), ((), ())),
        preferred_element_type=jnp.float32)
    d2sq = jnp.sum(d2b * d2b, axis=1, keepdims=True)
    s = (d2sq + d1sq_ref[...]) + mm

    s4 = s.reshape(blk // 8, 8, n1)
    bmin8 = jnp.min(s4, axis=0)
    r8 = jnp.argmin(s4, axis=0).astype(jnp.int32)
    sub = jax.lax.broadcasted_iota(jnp.int32, (8, n1), 0)
    idx8 = r8 * 8 + sub + i * _BLK
    big = jnp.int32(2**31 - 1)
    bmin = jnp.min(bmin8, axis=0, keepdims=True)
    bidx = jnp.min(jnp.where(bmin8 == bmin, idx8, big), axis=0,
                   keepdims=True)

    @pl.when(i == 0)
    def _():
        minval_ref[...] = bmin
        minidx_ref[...] = bidx

    @pl.when(i > 0)
    def _():
        better = bmin < minval_ref[...]
        minval_ref[...] = jnp.where(better, bmin, minval_ref[...])
        minidx_ref[...] = jnp.where(better, bidx, minidx_ref[...])

    @pl.when(i == n_blocks - 1)
    def _():
        d = jnp.sqrt(jnp.clip(minval_ref[...], 0.0, None))
        dist_ref[...] = jnp.transpose(d, (1, 0))
        it = jnp.transpose(minidx_ref[...], (1, 0))
        rows = jax.lax.broadcasted_iota(jnp.int32, (n1, 2), 0)
        cols = jax.lax.broadcasted_iota(jnp.int32, (n1, 2), 1)
        idx_ref[...] = jnp.where(cols == 0, rows, it)


def kernel(desc1, desc2):
    n1, dim = desc1.shape
    n2 = desc2.shape[0]
    assert n2 % _BLK == 0
    n_blocks = n2 // _BLK

    d1t = desc1.T * jnp.float32(-2.0)
    d1sq = jnp.sum(desc1 ** 2, axis=1)[None, :]

    match_dists, matches_idxs = pl.pallas_call(
        functools.partial(_nn_kernel, n_blocks),
        grid=(n_blocks,),
        in_specs=[
            pl.BlockSpec((dim, n1), lambda i: (0, 0)),
            pl.BlockSpec((_BLK, dim), lambda i: (i, 0)),
            pl.BlockSpec((1, n1), lambda i: (0, 0)),
        ],
        out_specs=[
            pl.BlockSpec((n1, 1), lambda i: (0, 0)),
            pl.BlockSpec((n1, 2), lambda i: (0, 0)),
        ],
        out_shape=[
            jax.ShapeDtypeStruct((n1, 1), jnp.float32),
            jax.ShapeDtypeStruct((n1, 2), jnp.int32),
        ],
        scratch_shapes=[
            pltpu.VMEM((1, n1), jnp.float32),
            pltpu.VMEM((1, n1), jnp.int32),
        ],
        compiler_params=pltpu.CompilerParams(
            dimension_semantics=("arbitrary",)),
    )(d1t, desc2, d1sq)

    return match_dists, matches_idxs


# speedup vs baseline: 2.0707x; 1.0141x over previous
import functools

import jax
import jax.numpy as jnp
from jax.experimental import pallas as pl
from jax.experimental.pallas import tpu as pltpu

_BLK = 10000
_NCORES = 2


def _nn_kernel(nb2, d1t_ref, d2_ref, d1sq_ref, val_ref, idx_ref,
               minval_ref, minidx_ref):
    p = pl.program_id(0)
    i = pl.program_id(1)
    d1t = d1t_ref[...]
    d2b = d2_ref[...]
    n1 = d1t.shape[1]
    blk = d2b.shape[0]

    mm = jax.lax.dot_general(
        d2b, d1t, (((1,), (0,)), ((), ())),
        preferred_element_type=jnp.float32)
    d2sq = jnp.sum(d2b * d2b, axis=1, keepdims=True)
    s = (d2sq + d1sq_ref[...]) + mm

    s4 = s.reshape(blk // 8, 8, n1)
    bmin8 = jnp.min(s4, axis=0)
    r8 = jnp.argmin(s4, axis=0).astype(jnp.int32)
    sub = jax.lax.broadcasted_iota(jnp.int32, (8, n1), 0)
    idx8 = r8 * 8 + sub + (p * nb2 + i) * _BLK
    big = jnp.int32(2**31 - 1)
    bmin = jnp.min(bmin8, axis=0, keepdims=True)
    bidx = jnp.min(jnp.where(bmin8 == bmin, idx8, big), axis=0,
                   keepdims=True)

    @pl.when(i == 0)
    def _():
        minval_ref[...] = bmin
        minidx_ref[...] = bidx

    @pl.when(i > 0)
    def _():
        better = bmin < minval_ref[...]
        minval_ref[...] = jnp.where(better, bmin, minval_ref[...])
        minidx_ref[...] = jnp.where(better, bidx, minidx_ref[...])

    @pl.when(i == nb2 - 1)
    def _():
        val_ref[...] = minval_ref[...][None]
        idx_ref[...] = minidx_ref[...][None]


def kernel(desc1, desc2):
    n1, dim = desc1.shape
    n2 = desc2.shape[0]
    n_blocks = n2 // _BLK
    nb2 = n_blocks // _NCORES

    d1t = desc1.T * jnp.float32(-2.0)
    d1sq = jnp.sum(desc1 ** 2, axis=1)[None, :]

    vals, idxs = pl.pallas_call(
        functools.partial(_nn_kernel, nb2),
        grid=(_NCORES, nb2),
        in_specs=[
            pl.BlockSpec((dim, n1), lambda p, i: (0, 0)),
            pl.BlockSpec((_BLK, dim), lambda p, i: (p * nb2 + i, 0)),
            pl.BlockSpec((1, n1), lambda p, i: (0, 0)),
        ],
        out_specs=[
            pl.BlockSpec((1, 1, n1), lambda p, i: (p, 0, 0)),
            pl.BlockSpec((1, 1, n1), lambda p, i: (p, 0, 0)),
        ],
        out_shape=[
            jax.ShapeDtypeStruct((_NCORES, 1, n1), jnp.float32),
            jax.ShapeDtypeStruct((_NCORES, 1, n1), jnp.int32),
        ],
        scratch_shapes=[
            pltpu.VMEM((1, n1), jnp.float32),
            pltpu.VMEM((1, n1), jnp.int32),
        ],
        compiler_params=pltpu.CompilerParams(
            dimension_semantics=("parallel", "arbitrary")),
    )(d1t, desc2, d1sq)

    v0, v1 = vals[0, 0], vals[1, 0]
    i0, i1 = idxs[0, 0], idxs[1, 0]
    take1 = v1 < v0
    val = jnp.where(take1, v1, v0)
    idx = jnp.where(take1, i1, i0)
    match_dists = jnp.sqrt(jnp.clip(val, 0.0, None)).reshape(n1, 1)
    rows = jnp.arange(n1, dtype=jnp.int32)[:, None]
    matches_idxs = jnp.concatenate([rows, idx.reshape(n1, 1)], axis=1)
    return match_dists, matches_idxs


# speedup vs baseline: 2.0964x; 1.0124x over previous
import functools

import jax
import jax.numpy as jnp
from jax.experimental import pallas as pl
from jax.experimental.pallas import tpu as pltpu

_BLK = 10000


def _nn_kernel(n_blocks, d1t_ref, d2_ref, d1sq_ref, dist_ref,
               idx_ref, minval_ref, minidx_ref):
    i = pl.program_id(0)
    d1t = d1t_ref[...]
    d2b = d2_ref[...]
    n1 = d1t.shape[1]
    blk = d2b.shape[0]

    mm = jax.lax.dot_general(
        d2b, d1t, (((1,), (0,)), ((), ())),
        preferred_element_type=jnp.float32)
    d2sq = jnp.sum(d2b * d2b, axis=1, keepdims=True)
    s = (d2sq + d1sq_ref[...]) + mm

    s4 = s.reshape(blk // 8, 8, n1)
    bmin8 = jnp.min(s4, axis=0)
    r8 = jnp.argmin(s4, axis=0).astype(jnp.int32)
    sub = jax.lax.broadcasted_iota(jnp.int32, (8, n1), 0)
    idx8 = r8 * 8 + sub + i * _BLK
    big = jnp.int32(2**31 - 1)
    bmin = jnp.min(bmin8, axis=0, keepdims=True)
    bidx = jnp.min(jnp.where(bmin8 == bmin, idx8, big), axis=0,
                   keepdims=True)

    @pl.when(i == 0)
    def _():
        minval_ref[...] = bmin
        minidx_ref[...] = bidx

    @pl.when(i > 0)
    def _():
        better = bmin < minval_ref[...]
        minval_ref[...] = jnp.where(better, bmin, minval_ref[...])
        minidx_ref[...] = jnp.where(better, bidx, minidx_ref[...])

    @pl.when(i == n_blocks - 1)
    def _():
        dist_ref[...] = jnp.sqrt(jnp.clip(minval_ref[...], 0.0, None))
        idx_ref[...] = minidx_ref[...]


def kernel(desc1, desc2):
    n1, dim = desc1.shape
    n2 = desc2.shape[0]
    assert n2 % _BLK == 0
    n_blocks = n2 // _BLK

    d1t = desc1.T * jnp.float32(-2.0)
    d1sq = jnp.sum(desc1 ** 2, axis=1)[None, :]

    dists_t, idxs_t = pl.pallas_call(
        functools.partial(_nn_kernel, n_blocks),
        grid=(n_blocks,),
        in_specs=[
            pl.BlockSpec((dim, n1), lambda i: (0, 0)),
            pl.BlockSpec((_BLK, dim), lambda i: (i, 0)),
            pl.BlockSpec((1, n1), lambda i: (0, 0)),
        ],
        out_specs=[
            pl.BlockSpec((1, n1), lambda i: (0, 0)),
            pl.BlockSpec((1, n1), lambda i: (0, 0)),
        ],
        out_shape=[
            jax.ShapeDtypeStruct((1, n1), jnp.float32),
            jax.ShapeDtypeStruct((1, n1), jnp.int32),
        ],
        scratch_shapes=[
            pltpu.VMEM((1, n1), jnp.float32),
            pltpu.VMEM((1, n1), jnp.int32),
        ],
        compiler_params=pltpu.CompilerParams(
            dimension_semantics=("arbitrary",)),
    )(d1t, desc2, d1sq)

    match_dists = dists_t.reshape(n1, 1)
    rows = jnp.arange(n1, dtype=jnp.int32)[:, None]
    matches_idxs = jnp.concatenate([rows, idxs_t.reshape(n1, 1)], axis=1)
    return match_dists, matches_idxs
